# baseline (speedup 1.0000x reference)
import functools

import jax
import jax.numpy as jnp
from jax import lax
from jax.experimental import pallas as pl
from jax.experimental.pallas import tpu as pltpu
from jax.experimental.pallas import tpu_sc as plsc

D_MODEL = 2048
HIDDEN = 2048
N_EXP = 8
TOPK = 2
T_TOK = 2048
T_BLK = 256
BT = 256
CAP = 2048
CAP_BLKS = CAP // BT
NROWS = N_EXP * CAP
NB = N_EXP + (TOPK * T_TOK) // BT
H_BLK = HIDDEN // 2

SC_NC = 2
SC_NS = 16
SC_NW = SC_NC * SC_NS


def _schedule_rows(cnt_col):
    cnt_i = cnt_col.astype(jnp.int32)
    nb_i = (cnt_i + (BT - 1)) >> 8
    ei = lax.broadcasted_iota(jnp.int32, (N_EXP, N_EXP), 0)
    ej = lax.broadcasted_iota(jnp.int32, (N_EXP, N_EXP), 1)
    tril = (ej < ei).astype(jnp.float32)
    start_f = lax.dot_general(tril, nb_i.astype(jnp.float32),
                              (((1,), (0,)), ((), ())),
                              preferred_element_type=jnp.float32)
    start_i = start_f.astype(jnp.int32)
    total_i = jnp.sum(nb_i, keepdims=True)
    s_io = lax.broadcasted_iota(jnp.int32, (1, 128), 1)
    cmp = (start_i <= s_io).astype(jnp.int32)
    e_raw = jnp.clip(jnp.sum(cmp, axis=0, keepdims=True) - 1, 0, N_EXP - 1)
    emask = lax.broadcasted_iota(jnp.int32, (N_EXP, 128), 0) == e_raw
    start_sel = jnp.sum(jnp.where(emask, start_i, 0), axis=0, keepdims=True)
    r_raw = e_raw * CAP_BLKS + (s_io - start_sel)
    valid = s_io < total_i
    lmask = s_io == (total_i - 1)
    e_last = jnp.sum(jnp.where(lmask, e_raw, 0), axis=1, keepdims=True)
    r_last = jnp.sum(jnp.where(lmask, r_raw, 0), axis=1, keepdims=True)
    e_s = jnp.where(valid, e_raw, e_last)
    r_s = jnp.where(valid, r_raw, r_last)
    v_s = valid.astype(jnp.int32)
    z = jnp.zeros((N_EXP - 3, 128), jnp.int32)
    return jnp.concatenate([e_s, r_s, v_s, z], axis=0)


def _route_body(x_ref, wg_ref, pos_ref, w_ref, sched_ref, xbf_ref, cnt_s):
    t = pl.program_id(0)

    @pl.when(t == 0)
    def _():
        cnt_s[...] = jnp.zeros_like(cnt_s)

    x_bf = x_ref[...].astype(jnp.bfloat16)
    xbf_ref[...] = x_bf
    logits = lax.dot_general(
        wg_ref[...].astype(jnp.bfloat16), x_bf,
        (((1,), (1,)), ((), ())), preferred_element_type=jnp.float32)
    eidx = lax.broadcasted_iota(jnp.int32, logits.shape, 0)
    m1 = jnp.max(logits, axis=0, keepdims=True)
    a1 = jnp.min(jnp.where(logits == m1, eidx, N_EXP), axis=0, keepdims=True)
    oh0 = (eidx == a1).astype(jnp.float32)
    masked = jnp.where(eidx == a1, -jnp.inf, logits)
    m2 = jnp.max(masked, axis=0, keepdims=True)
    a2 = jnp.min(jnp.where(masked == m2, eidx, N_EXP), axis=0, keepdims=True)
    oh1 = (eidx == a2).astype(jnp.float32)
    e2 = jnp.exp(m2 - m1)
    w1v = 1.0 / (1.0 + e2)
    w2v = 1.0 - w1v

    ti = lax.broadcasted_iota(jnp.int32, (T_BLK, T_BLK), 0)
    tj = lax.broadcasted_iota(jnp.int32, (T_BLK, T_BLK), 1)
    tri = (ti < tj).astype(jnp.float32)
    oh01 = oh0 + oh1
    c01 = lax.dot_general(oh01, tri, (((1,), (0,)), ((), ())),
                          preferred_element_type=jnp.float32)

    lane0 = (lax.broadcasted_iota(jnp.int32, cnt_s.shape, 1) == 0)
    cnt_col = jnp.sum(jnp.where(lane0, cnt_s[...], 0.0), axis=1,
                      keepdims=True)
    r = c01 + cnt_col
    pos0 = jnp.sum(oh0 * r, axis=0, keepdims=True).astype(jnp.int32) + a1 * CAP
    pos1 = jnp.sum(oh1 * r, axis=0, keepdims=True).astype(jnp.int32) + a2 * CAP

    new_cnt = cnt_s[...] + jnp.sum(oh01, axis=1, keepdims=True) * lane0
    cnt_s[...] = new_cnt

    zi = jnp.zeros((N_EXP - 2, T_BLK), jnp.int32)
    pos_ref[...] = jnp.concatenate([pos0, pos1, zi], axis=0)
    zf = jnp.zeros((N_EXP - 2, T_BLK), jnp.float32)
    w_ref[...] = jnp.concatenate([w1v, w2v, zf], axis=0)

    @pl.when(t == pl.num_programs(0) - 1)
    def _():
        cnt_col = jnp.sum(jnp.where(lane0, new_cnt, 0.0), axis=1,
                          keepdims=True)
        sched_ref[...] = _schedule_rows(cnt_col)


def _route(x2, Wg):
    return pl.pallas_call(
        _route_body,
        grid=(T_TOK // T_BLK,),
        in_specs=[
            pl.BlockSpec((T_BLK, D_MODEL), lambda t: (t, 0)),
            pl.BlockSpec((N_EXP, D_MODEL), lambda t: (0, 0)),
        ],
        out_specs=[
            pl.BlockSpec((N_EXP, T_BLK), lambda t: (0, t)),
            pl.BlockSpec((N_EXP, T_BLK), lambda t: (0, t)),
            pl.BlockSpec((N_EXP, 128), lambda t: (0, 0)),
            pl.BlockSpec((T_BLK, D_MODEL), lambda t: (t, 0)),
        ],
        out_shape=[
            jax.ShapeDtypeStruct((N_EXP, T_TOK), jnp.int32),
            jax.ShapeDtypeStruct((N_EXP, T_TOK), jnp.float32),
            jax.ShapeDtypeStruct((N_EXP, 128), jnp.int32),
            jax.ShapeDtypeStruct((T_TOK, D_MODEL), jnp.bfloat16),
        ],
        scratch_shapes=[pltpu.VMEM((N_EXP, 128), jnp.float32)],
    )(x2, Wg)


_DISP_CH = 16


_DISP_NCH = (T_TOK // SC_NW) // _DISP_CH


@functools.lru_cache(maxsize=1)
def _make_dispatch_sc():
    @functools.partial(
        pl.kernel,
        out_type=jax.ShapeDtypeStruct((NROWS, D_MODEL // 2), jnp.int32),
        mesh=plsc.VectorSubcoreMesh(core_axis_name="c", subcore_axis_name="s"),
        scratch_types=[
            pltpu.VMEM((_DISP_CH, D_MODEL // 2), jnp.int32),
            pltpu.VMEM((_DISP_CH, D_MODEL // 2), jnp.int32),
            pltpu.VMEM((2, _DISP_NCH, _DISP_CH), jnp.int32),
            pltpu.SemaphoreType.DMA,
            pltpu.SemaphoreType.DMA,
            pltpu.SemaphoreType.DMA,
        ],
    )
    def disp(x_hbm, pos3_hbm, xg_hbm, xr0, xr1, idxb, seml, sems0, sems1):
        wid = lax.axis_index("s") * SC_NC + lax.axis_index("c")
        base = wid * (T_TOK // SC_NW)
        pltpu.sync_copy(pos3_hbm.at[0, pl.ds(wid * _DISP_NCH, _DISP_NCH)],
                        idxb.at[0])
        pltpu.sync_copy(pos3_hbm.at[1, pl.ds(wid * _DISP_NCH, _DISP_NCH)],
                        idxb.at[1])
        pltpu.sync_copy(x_hbm.at[pl.ds(base, _DISP_CH)], xr0)
        for c in range(_DISP_NCH):
            cur = xr0 if c % 2 == 0 else xr1
            nxt = xr1 if c % 2 == 0 else xr0
            ld = None
            if c + 1 < _DISP_NCH:
                ld = pltpu.async_copy(
                    x_hbm.at[pl.ds(base + (c + 1) * _DISP_CH, _DISP_CH)],
                    nxt, seml)
            s0 = pltpu.async_copy(cur, xg_hbm.at[idxb.at[0, c]], sems0)
            s1 = pltpu.async_copy(cur, xg_hbm.at[idxb.at[1, c]], sems1)
            s0.wait()
            s1.wait()
            if ld is not None:
                ld.wait()

    return disp


def _dispatch_sc(xbf, pos):
    x32 = lax.bitcast_convert_type(
        xbf.reshape(T_TOK, D_MODEL // 2, 2), jnp.int32)
    pos3 = pos[:2].reshape(2, T_TOK // _DISP_CH, _DISP_CH)
    xg32 = _make_dispatch_sc()(x32, pos3)
    return lax.bitcast_convert_type(xg32, jnp.bfloat16).reshape(
        NROWS, D_MODEL)


def _ffn_a_body(sc_ref, xg_ref, w1_ref, b1_ref, h_ref):
    s = pl.program_id(0)

    @pl.when(sc_ref[2, s] == 1)
    def _():
        h = lax.dot_general(xg_ref[...], w1_ref[0].astype(jnp.bfloat16),
                            (((1,), (1,)), ((), ())),
                            preferred_element_type=jnp.float32)
        h_ref[...] = jnp.maximum(h + b1_ref[0], 0.0).astype(jnp.bfloat16)


def _ffn_b_body(sc_ref, h_ref, w2_ref, b2_ref, y_ref):
    s = pl.program_id(0)

    @pl.when(sc_ref[2, s] == 1)
    def _():
        yp = lax.dot_general(h_ref[...], w2_ref[0].astype(jnp.bfloat16),
                             (((1,), (1,)), ((), ())),
                             preferred_element_type=jnp.float32)
        y_ref[...] = yp + b2_ref[0]


def _ffn(xg, W1, b1, W2, b2, sched):
    h = pl.pallas_call(
        _ffn_a_body,
        grid_spec=pltpu.PrefetchScalarGridSpec(
            num_scalar_prefetch=1,
            grid=(NB,),
            in_specs=[
                pl.BlockSpec((BT, D_MODEL), lambda s, sc: (sc[1, s], 0)),
                pl.BlockSpec((1, HIDDEN, D_MODEL), lambda s, sc: (sc[0, s], 0, 0)),
                pl.BlockSpec((1, 1, HIDDEN), lambda s, sc: (sc[0, s], 0, 0)),
            ],
            out_specs=pl.BlockSpec((BT, HIDDEN), lambda s, sc: (sc[1, s], 0)),
        ),
        out_shape=jax.ShapeDtypeStruct((NROWS, HIDDEN), jnp.bfloat16),
        compiler_params=pltpu.CompilerParams(
            dimension_semantics=("arbitrary",)),
    )(sched, xg, W1, b1.reshape(N_EXP, 1, HIDDEN))
    return pl.pallas_call(
        _ffn_b_body,
        grid_spec=pltpu.PrefetchScalarGridSpec(
            num_scalar_prefetch=1,
            grid=(NB,),
            in_specs=[
                pl.BlockSpec((BT, HIDDEN), lambda s, sc: (sc[1, s], 0)),
                pl.BlockSpec((1, D_MODEL, HIDDEN), lambda s, sc: (sc[0, s], 0, 0)),
                pl.BlockSpec((1, 1, D_MODEL), lambda s, sc: (sc[0, s], 0, 0)),
            ],
            out_specs=pl.BlockSpec((BT, D_MODEL), lambda s, sc: (sc[1, s], 0)),
        ),
        out_shape=jax.ShapeDtypeStruct((NROWS, D_MODEL), jnp.float32),
        compiler_params=pltpu.CompilerParams(
            dimension_semantics=("arbitrary",)),
    )(sched, h, W2, b2.reshape(N_EXP, 1, D_MODEL))


_COMB_CH = 16
_COMB_PW = T_TOK // SC_NW
_COMB_NCH = _COMB_PW // _COMB_CH


@functools.lru_cache(maxsize=1)
def _make_combine_sc():
    @functools.partial(
        pl.kernel,
        out_type=jax.ShapeDtypeStruct((T_TOK, D_MODEL), jnp.float32),
        mesh=plsc.VectorSubcoreMesh(core_axis_name="c", subcore_axis_name="s"),
        scratch_types=[
            pltpu.VMEM((_COMB_CH, D_MODEL), jnp.float32),
            pltpu.VMEM((_COMB_CH, D_MODEL), jnp.float32),
            pltpu.VMEM((_COMB_CH, D_MODEL), jnp.float32),
            pltpu.VMEM((_COMB_PW,), jnp.int32),
            pltpu.VMEM((_COMB_PW,), jnp.int32),
            pltpu.VMEM((_COMB_PW,), jnp.float32),
            pltpu.VMEM((_COMB_PW,), jnp.float32),
            pltpu.SemaphoreType.DMA,
            pltpu.SemaphoreType.DMA,
        ],
    )
    def comb(y_hbm, pos_hbm, w_hbm, out_hbm, y0buf, y1buf, obuf,
             idx0a, idx1a, wv0a, wv1a, sem0, sem1):
        wid = lax.axis_index("s") * SC_NC + lax.axis_index("c")
        base = wid * _COMB_PW
        pltpu.sync_copy(pos_hbm.at[0, pl.ds(base, _COMB_PW)], idx0a)
        pltpu.sync_copy(pos_hbm.at[1, pl.ds(base, _COMB_PW)], idx1a)
        pltpu.sync_copy(w_hbm.at[0, pl.ds(base, _COMB_PW)], wv0a)
        pltpu.sync_copy(w_hbm.at[1, pl.ds(base, _COMB_PW)], wv1a)
        dn = lax.GatherDimensionNumbers(
            offset_dims=(), collapsed_slice_dims=(0,), start_index_map=(0,))

        def chunk(c, carry):
            coff = c * _COMB_CH
            g0 = pltpu.async_copy(
                y_hbm.at[idx0a.at[pl.ds(coff, _COMB_CH)]], y0buf, sem0)
            g1 = pltpu.async_copy(
                y_hbm.at[idx1a.at[pl.ds(coff, _COMB_CH)]], y1buf, sem1)
            g0.wait()
            g1.wait()
            wc0 = wv0a[pl.ds(coff, 16)]
            wc1 = wv1a[pl.ds(coff, 16)]
            for t in range(_COMB_CH):
                tt = jnp.full((16, 1), t, jnp.int32)
                w0v = lax.gather(wc0, tt, dn, (1,),
                                 mode=lax.GatherScatterMode.PROMISE_IN_BOUNDS)
                w1v = lax.gather(wc1, tt, dn, (1,),
                                 mode=lax.GatherScatterMode.PROMISE_IN_BOUNDS)

                def vec(j, carry2):
                    for u in range(8):
                        sl = pl.ds(j * 128 + u * 16, 16)
                        obuf[t, sl] = w0v * y0buf[t, sl] + w1v * y1buf[t, sl]
                    return carry2

                lax.fori_loop(0, D_MODEL // 128, vec, 0)
            pltpu.sync_copy(obuf, out_hbm.at[pl.ds(base + coff, _COMB_CH)])
            return carry

        lax.fori_loop(0, _COMB_NCH, chunk, 0)

    return comb


def _combine_sc(y, pos, w):
    return _make_combine_sc()(y, pos, w)


def kernel(x, Wg, W1, b1, W2, b2):
    x2 = x.reshape(-1, x.shape[-1])
    pos, w, sched, xbf = _route(x2, Wg)
    xg = _dispatch_sc(xbf, pos)
    y = _ffn(xg, W1, b1, W2, b2, sched)
    out = _combine_sc(y, pos, w)
    return out.reshape(x.shape)


# speedup vs baseline: 3.1176x; 3.1176x over previous
import functools

import jax
import jax.numpy as jnp
from jax import lax
from jax.experimental import pallas as pl
from jax.experimental.pallas import tpu as pltpu
from jax.experimental.pallas import tpu_sc as plsc

D_MODEL = 2048
HIDDEN = 2048
N_EXP = 8
TOPK = 2
T_TOK = 2048
T_BLK = 256
BT = 256
CAP = 2048
CAP_BLKS = CAP // BT
NROWS = N_EXP * CAP
NB = N_EXP + (TOPK * T_TOK) // BT
H_BLK = HIDDEN // 2

SC_NC = 2
SC_NS = 16
SC_NW = SC_NC * SC_NS


def _schedule_rows(cnt_col):
    cnt_i = cnt_col.astype(jnp.int32)
    nb_i = (cnt_i + (BT - 1)) >> 8
    ei = lax.broadcasted_iota(jnp.int32, (N_EXP, N_EXP), 0)
    ej = lax.broadcasted_iota(jnp.int32, (N_EXP, N_EXP), 1)
    tril = (ej < ei).astype(jnp.float32)
    start_f = lax.dot_general(tril, nb_i.astype(jnp.float32),
                              (((1,), (0,)), ((), ())),
                              preferred_element_type=jnp.float32)
    start_i = start_f.astype(jnp.int32)
    total_i = jnp.sum(nb_i, keepdims=True)
    s_io = lax.broadcasted_iota(jnp.int32, (1, 128), 1)
    cmp = (start_i <= s_io).astype(jnp.int32)
    e_raw = jnp.clip(jnp.sum(cmp, axis=0, keepdims=True) - 1, 0, N_EXP - 1)
    emask = lax.broadcasted_iota(jnp.int32, (N_EXP, 128), 0) == e_raw
    start_sel = jnp.sum(jnp.where(emask, start_i, 0), axis=0, keepdims=True)
    r_raw = e_raw * CAP_BLKS + (s_io - start_sel)
    valid = s_io < total_i
    lmask = s_io == (total_i - 1)
    e_last = jnp.sum(jnp.where(lmask, e_raw, 0), axis=1, keepdims=True)
    r_last = jnp.sum(jnp.where(lmask, r_raw, 0), axis=1, keepdims=True)
    e_s = jnp.where(valid, e_raw, e_last)
    r_s = jnp.where(valid, r_raw, r_last)
    v_s = valid.astype(jnp.int32)
    z = jnp.zeros((N_EXP - 3, 128), jnp.int32)
    return jnp.concatenate([e_s, r_s, v_s, z], axis=0)


def _route_body(x_ref, wg_ref, pos_ref, w_ref, sched_ref, cnt_s):
    t = pl.program_id(0)

    @pl.when(t == 0)
    def _():
        cnt_s[...] = jnp.zeros_like(cnt_s)

    logits = lax.dot_general(
        wg_ref[...].astype(jnp.bfloat16), x_ref[...].astype(jnp.bfloat16),
        (((1,), (1,)), ((), ())), preferred_element_type=jnp.float32)
    eidx = lax.broadcasted_iota(jnp.int32, logits.shape, 0)
    m1 = jnp.max(logits, axis=0, keepdims=True)
    a1 = jnp.min(jnp.where(logits == m1, eidx, N_EXP), axis=0, keepdims=True)
    oh0 = (eidx == a1).astype(jnp.float32)
    masked = jnp.where(eidx == a1, -jnp.inf, logits)
    m2 = jnp.max(masked, axis=0, keepdims=True)
    a2 = jnp.min(jnp.where(masked == m2, eidx, N_EXP), axis=0, keepdims=True)
    oh1 = (eidx == a2).astype(jnp.float32)
    e2 = jnp.exp(m2 - m1)
    w1v = 1.0 / (1.0 + e2)
    w2v = 1.0 - w1v

    ti = lax.broadcasted_iota(jnp.int32, (T_BLK, T_BLK), 0)
    tj = lax.broadcasted_iota(jnp.int32, (T_BLK, T_BLK), 1)
    tri = (ti < tj).astype(jnp.float32)
    oh01 = oh0 + oh1
    c01 = lax.dot_general(oh01, tri, (((1,), (0,)), ((), ())),
                          preferred_element_type=jnp.float32)

    lane0 = (lax.broadcasted_iota(jnp.int32, cnt_s.shape, 1) == 0)
    cnt_col = jnp.sum(jnp.where(lane0, cnt_s[...], 0.0), axis=1,
                      keepdims=True)
    r = c01 + cnt_col
    pos0 = jnp.sum(oh0 * r, axis=0, keepdims=True).astype(jnp.int32) + a1 * CAP
    pos1 = jnp.sum(oh1 * r, axis=0, keepdims=True).astype(jnp.int32) + a2 * CAP

    new_cnt = cnt_s[...] + jnp.sum(oh01, axis=1, keepdims=True) * lane0
    cnt_s[...] = new_cnt

    zi = jnp.zeros((N_EXP - 2, T_BLK), jnp.int32)
    pos_ref[...] = jnp.concatenate([pos0, pos1, zi], axis=0)
    zf = jnp.zeros((N_EXP - 2, T_BLK), jnp.float32)
    w_ref[...] = jnp.concatenate([w1v, w2v, zf], axis=0)

    @pl.when(t == pl.num_programs(0) - 1)
    def _():
        cnt_col = jnp.sum(jnp.where(lane0, new_cnt, 0.0), axis=1,
                          keepdims=True)
        sched_ref[...] = _schedule_rows(cnt_col)


def _route(x2, Wg):
    return pl.pallas_call(
        _route_body,
        grid=(T_TOK // T_BLK,),
        in_specs=[
            pl.BlockSpec((T_BLK, D_MODEL), lambda t: (t, 0)),
            pl.BlockSpec((N_EXP, D_MODEL), lambda t: (0, 0)),
        ],
        out_specs=[
            pl.BlockSpec((N_EXP, T_BLK), lambda t: (0, t)),
            pl.BlockSpec((N_EXP, T_BLK), lambda t: (0, t)),
            pl.BlockSpec((N_EXP, 128), lambda t: (0, 0)),
        ],
        out_shape=[
            jax.ShapeDtypeStruct((N_EXP, T_TOK), jnp.int32),
            jax.ShapeDtypeStruct((N_EXP, T_TOK), jnp.float32),
            jax.ShapeDtypeStruct((N_EXP, 128), jnp.int32),
        ],
        scratch_shapes=[pltpu.VMEM((N_EXP, 128), jnp.float32)],
    )(x2, Wg)


_DISP_CH = 16


_DISP_NCH = (T_TOK // SC_NW) // _DISP_CH


@functools.lru_cache(maxsize=1)
def _make_dispatch_sc():
    @functools.partial(
        pl.kernel,
        out_type=jax.ShapeDtypeStruct((NROWS, D_MODEL), jnp.float32),
        mesh=plsc.VectorSubcoreMesh(core_axis_name="c", subcore_axis_name="s"),
        scratch_types=[
            pltpu.VMEM((_DISP_CH, D_MODEL), jnp.float32),
            pltpu.VMEM((_DISP_CH, D_MODEL), jnp.float32),
            pltpu.VMEM((2, _DISP_NCH, _DISP_CH), jnp.int32),
            pltpu.SemaphoreType.DMA,
            pltpu.SemaphoreType.DMA,
            pltpu.SemaphoreType.DMA,
        ],
    )
    def disp(x_hbm, pos3_hbm, xg_hbm, xr0, xr1, idxb, seml, sems0, sems1):
        wid = lax.axis_index("s") * SC_NC + lax.axis_index("c")
        base = wid * (T_TOK // SC_NW)
        pltpu.sync_copy(pos3_hbm.at[0, pl.ds(wid * _DISP_NCH, _DISP_NCH)],
                        idxb.at[0])
        pltpu.sync_copy(pos3_hbm.at[1, pl.ds(wid * _DISP_NCH, _DISP_NCH)],
                        idxb.at[1])
        pltpu.sync_copy(x_hbm.at[pl.ds(base, _DISP_CH)], xr0)
        for c in range(_DISP_NCH):
            cur = xr0 if c % 2 == 0 else xr1
            nxt = xr1 if c % 2 == 0 else xr0
            ld = None
            if c + 1 < _DISP_NCH:
                ld = pltpu.async_copy(
                    x_hbm.at[pl.ds(base + (c + 1) * _DISP_CH, _DISP_CH)],
                    nxt, seml)
            s0 = pltpu.async_copy(cur, xg_hbm.at[idxb.at[0, c]], sems0)
            s1 = pltpu.async_copy(cur, xg_hbm.at[idxb.at[1, c]], sems1)
            s0.wait()
            s1.wait()
            if ld is not None:
                ld.wait()

    return disp


def _dispatch_sc(x2, pos):
    pos3 = pos[:2].reshape(2, T_TOK // _DISP_CH, _DISP_CH)
    return _make_dispatch_sc()(x2, pos3)


def _ffn_a_body(sc_ref, xg_ref, w1_ref, b1_ref, h_ref):
    s = pl.program_id(0)

    @pl.when(sc_ref[2, s] == 1)
    def _():
        h = lax.dot_general(xg_ref[...], w1_ref[0], (((1,), (1,)), ((), ())),
                            preferred_element_type=jnp.float32)
        h_ref[...] = jnp.maximum(h + b1_ref[0], 0.0).astype(jnp.bfloat16)


def _ffn_b_body(sc_ref, h_ref, w2_ref, b2_ref, y_ref):
    s = pl.program_id(0)

    @pl.when(sc_ref[2, s] == 1)
    def _():
        yp = lax.dot_general(h_ref[...], w2_ref[0].astype(jnp.bfloat16),
                             (((1,), (1,)), ((), ())),
                             preferred_element_type=jnp.float32)
        y_ref[...] = yp + b2_ref[0]


def _ffn(xg, W1, b1, W2, b2, sched):
    h = pl.pallas_call(
        _ffn_a_body,
        grid_spec=pltpu.PrefetchScalarGridSpec(
            num_scalar_prefetch=1,
            grid=(NB,),
            in_specs=[
                pl.BlockSpec((BT, D_MODEL), lambda s, sc: (sc[1, s], 0)),
                pl.BlockSpec((1, HIDDEN, D_MODEL), lambda s, sc: (sc[0, s], 0, 0)),
                pl.BlockSpec((1, 1, HIDDEN), lambda s, sc: (sc[0, s], 0, 0)),
            ],
            out_specs=pl.BlockSpec((BT, HIDDEN), lambda s, sc: (sc[1, s], 0)),
        ),
        out_shape=jax.ShapeDtypeStruct((NROWS, HIDDEN), jnp.bfloat16),
        compiler_params=pltpu.CompilerParams(
            dimension_semantics=("arbitrary",)),
    )(sched, xg, W1, b1.reshape(N_EXP, 1, HIDDEN))
    return pl.pallas_call(
        _ffn_b_body,
        grid_spec=pltpu.PrefetchScalarGridSpec(
            num_scalar_prefetch=1,
            grid=(NB,),
            in_specs=[
                pl.BlockSpec((BT, HIDDEN), lambda s, sc: (sc[1, s], 0)),
                pl.BlockSpec((1, D_MODEL, HIDDEN), lambda s, sc: (sc[0, s], 0, 0)),
                pl.BlockSpec((1, 1, D_MODEL), lambda s, sc: (sc[0, s], 0, 0)),
            ],
            out_specs=pl.BlockSpec((BT, D_MODEL), lambda s, sc: (sc[1, s], 0)),
        ),
        out_shape=jax.ShapeDtypeStruct((NROWS, D_MODEL), jnp.float32),
        compiler_params=pltpu.CompilerParams(
            dimension_semantics=("arbitrary",)),
    )(sched, h, W2, b2.reshape(N_EXP, 1, D_MODEL))


_COMB_CH = 8
_COMB_PW = T_TOK // SC_NW
_COMB_NCH = _COMB_PW // _COMB_CH


@functools.lru_cache(maxsize=1)
def _make_combine_sc():
    @functools.partial(
        pl.kernel,
        out_type=jax.ShapeDtypeStruct((T_TOK, D_MODEL), jnp.float32),
        mesh=plsc.VectorSubcoreMesh(core_axis_name="c", subcore_axis_name="s"),
        scratch_types=[
            pltpu.VMEM((2, _COMB_CH, D_MODEL), jnp.float32),
            pltpu.VMEM((2, _COMB_CH, D_MODEL), jnp.float32),
            pltpu.VMEM((_COMB_CH, D_MODEL), jnp.float32),
            pltpu.VMEM((_COMB_PW,), jnp.int32),
            pltpu.VMEM((_COMB_PW,), jnp.int32),
            pltpu.VMEM((_COMB_PW,), jnp.float32),
            pltpu.VMEM((_COMB_PW,), jnp.float32),
            pltpu.SemaphoreType.DMA,
            pltpu.SemaphoreType.DMA,
            pltpu.SemaphoreType.DMA,
            pltpu.SemaphoreType.DMA,
        ],
    )
    def comb(y_hbm, pos_hbm, w_hbm, out_hbm, y0buf, y1buf, obuf,
             idx0a, idx1a, wv0a, wv1a, sem0a, sem1a, sem0b, sem1b):
        wid = lax.axis_index("s") * SC_NC + lax.axis_index("c")
        base = wid * _COMB_PW
        pltpu.sync_copy(pos_hbm.at[0, pl.ds(base, _COMB_PW)], idx0a)
        pltpu.sync_copy(pos_hbm.at[1, pl.ds(base, _COMB_PW)], idx1a)
        pltpu.sync_copy(w_hbm.at[0, pl.ds(base, _COMB_PW)], wv0a)
        pltpu.sync_copy(w_hbm.at[1, pl.ds(base, _COMB_PW)], wv1a)
        dn = lax.GatherDimensionNumbers(
            offset_dims=(), collapsed_slice_dims=(0,), start_index_map=(0,))
        sems = ((sem0a, sem1a), (sem0b, sem1b))

        def gathers(c, s):
            coff = c * _COMB_CH
            g0 = pltpu.async_copy(
                y_hbm.at[idx0a.at[pl.ds(coff, _COMB_CH)]], y0buf.at[s],
                sems[s][0])
            g1 = pltpu.async_copy(
                y_hbm.at[idx1a.at[pl.ds(coff, _COMB_CH)]], y1buf.at[s],
                sems[s][1])
            return g0, g1

        pend = gathers(0, 0)
        for c in range(_COMB_NCH):
            s = c % 2
            pend[0].wait()
            pend[1].wait()
            if c + 1 < _COMB_NCH:
                pend = gathers(c + 1, 1 - s)
            coff = c * _COMB_CH
            w16 = (c // 2) * 16
            wc0 = wv0a[pl.ds(w16, 16)]
            wc1 = wv1a[pl.ds(w16, 16)]
            for t in range(_COMB_CH):
                ti = (c % 2) * _COMB_CH + t
                tt = jnp.full((16, 1), ti, jnp.int32)
                w0v = lax.gather(wc0, tt, dn, (1,),
                                 mode=lax.GatherScatterMode.PROMISE_IN_BOUNDS)
                w1v = lax.gather(wc1, tt, dn, (1,),
                                 mode=lax.GatherScatterMode.PROMISE_IN_BOUNDS)

                def vec(j, carry2, _t=t, _s=s, _w0=w0v, _w1=w1v):
                    for u in range(8):
                        sl = pl.ds(j * 128 + u * 16, 16)
                        obuf[_t, sl] = (_w0 * y0buf[_s, _t, sl]
                                        + _w1 * y1buf[_s, _t, sl])
                    return carry2

                lax.fori_loop(0, D_MODEL // 128, vec, 0)
            pltpu.sync_copy(obuf, out_hbm.at[pl.ds(base + coff, _COMB_CH)])

    return comb


def _combine_sc(y, pos, w):
    return _make_combine_sc()(y, pos, w)


def kernel(x, Wg, W1, b1, W2, b2):
    x2 = x.reshape(-1, x.shape[-1])
    pos, w, sched = _route(x2, Wg)
    xg = _dispatch_sc(x2, pos)
    y = _ffn(xg, W1, b1, W2, b2, sched)
    out = _combine_sc(y, pos, w)
    return out.reshape(x.shape)
